# baseline (speedup 1.0000x reference)
import functools

import jax
import jax.numpy as jnp
from jax import lax
from jax.experimental import pallas as pl
from jax.experimental.pallas import tpu as pltpu
from jax.experimental.pallas import tpu_sc as plsc

E = 8
K = 2
N = 2048
D = 768
F = 2048
T = 256
P = N * K
S = ((P + E * (T - 1) + T - 1) // T) * T
NB = S // T

NC, NS = 2, 16
NW = NC * NS
L = 16


def _router_body(x_ref, wr_ref, i_ref, w0_ref, w1_ref):
    logits = jnp.dot(x_ref[...], wr_ref[...], preferred_element_type=jnp.float32)
    m = jnp.max(logits, axis=-1, keepdims=True)
    p = jnp.exp(logits - m)
    p = p / jnp.sum(p, axis=-1, keepdims=True)
    iota = lax.broadcasted_iota(jnp.int32, p.shape, 1)
    m1 = jnp.max(p, axis=-1, keepdims=True)
    i1 = jnp.min(jnp.where(p == m1, iota, E), axis=-1, keepdims=True)
    p2 = jnp.where(iota == i1, -1.0, p)
    m2 = jnp.max(p2, axis=-1, keepdims=True)
    i2 = jnp.min(jnp.where(p2 == m2, iota, E), axis=-1, keepdims=True)
    s = m1 + m2
    i_ref[...] = jnp.concatenate([i1, i2], axis=1)
    w0_ref[...] = jnp.broadcast_to(m1 / s, (m1.shape[0], L))
    w1_ref[...] = jnp.broadcast_to(m2 / s, (m2.shape[0], L))


def _router(x_flat, Wr):
    return pl.pallas_call(
        _router_body,
        out_shape=(
            jax.ShapeDtypeStruct((N, K), jnp.int32),
            jax.ShapeDtypeStruct((N, L), jnp.float32),
            jax.ShapeDtypeStruct((N, L), jnp.float32),
        ),
    )(x_flat, Wr)


@functools.cache
def _sc_dispatch_kernel():
    mesh = plsc.VectorSubcoreMesh(
        core_axis_name="c", subcore_axis_name="s", num_cores=NC, num_subcores=NS
    )
    tok_per_w = N // NW

    @functools.partial(
        pl.kernel,
        mesh=mesh,
        out_type=jax.ShapeDtypeStruct((S, D), jnp.float32),
        scratch_types=[
            pltpu.VMEM((tok_per_w, D), jnp.float32),
            pltpu.VMEM((tok_per_w,), jnp.int32),
            pltpu.VMEM((tok_per_w,), jnp.int32),
            pltpu.SemaphoreType.DMA,
            pltpu.SemaphoreType.DMA,
        ],
    )
    def k(x_hbm, p0_hbm, p1_hbm, xs_hbm, xbuf, d0_v, d1_v, lsem, ssem):
        wid = lax.axis_index("s") * NC + lax.axis_index("c")
        tbase = pl.multiple_of(wid * tok_per_w, 8)
        lc = pltpu.async_copy(x_hbm.at[pl.ds(tbase, tok_per_w)], xbuf, lsem)
        pltpu.sync_copy(p0_hbm.at[pl.ds(tbase, tok_per_w)], d0_v)
        pltpu.sync_copy(p1_hbm.at[pl.ds(tbase, tok_per_w)], d1_v)
        lc.wait()
        s0 = pltpu.async_copy(xbuf, xs_hbm.at[d0_v], ssem)
        s1 = pltpu.async_copy(xbuf, xs_hbm.at[d1_v], ssem)
        s0.wait()
        s1.wait()

    return k


def _mlp_body(be_ref, xs_ref, w1_ref, b1_ref, w2_ref, b2_ref, ys_ref):
    i = pl.program_id(0)

    @pl.when(i < be_ref[NB])
    def _():
        h = jax.nn.gelu(
            jnp.dot(xs_ref[...], w1_ref[0], preferred_element_type=jnp.float32)
            + b1_ref[0]
        )
        y = jnp.dot(h, w2_ref[0], preferred_element_type=jnp.float32)
        ys_ref[...] = y + b2_ref[0]


def _grouped_mlp(block_expert, Xs, W1, b1, W2, b2):
    grid_spec = pltpu.PrefetchScalarGridSpec(
        num_scalar_prefetch=1,
        grid=(NB,),
        in_specs=[
            pl.BlockSpec((T, D), lambda i, be: (i, 0)),
            pl.BlockSpec((1, D, F), lambda i, be: (be[i], 0, 0)),
            pl.BlockSpec((1, 1, F), lambda i, be: (be[i], 0, 0)),
            pl.BlockSpec((1, F, D), lambda i, be: (be[i], 0, 0)),
            pl.BlockSpec((1, 1, D), lambda i, be: (be[i], 0, 0)),
        ],
        out_specs=pl.BlockSpec((T, D), lambda i, be: (i, 0)),
    )
    return pl.pallas_call(
        _mlp_body,
        grid_spec=grid_spec,
        out_shape=jax.ShapeDtypeStruct((S, D), jnp.float32),
    )(block_expert, Xs, W1, b1.reshape(E, 1, F), W2, b2.reshape(E, 1, D))


CCH = 32


@functools.cache
def _sc_combine_kernel():
    mesh = plsc.VectorSubcoreMesh(
        core_axis_name="c", subcore_axis_name="s", num_cores=NC, num_subcores=NS
    )
    tok_per_w = N // NW

    @functools.partial(
        pl.kernel,
        mesh=mesh,
        out_type=jax.ShapeDtypeStruct((N, D), jnp.float32),
        scratch_types=[
            pltpu.VMEM((CCH,), jnp.int32),
            pltpu.VMEM((CCH,), jnp.int32),
            pltpu.VMEM((CCH,), jnp.int32),
            pltpu.VMEM((CCH,), jnp.int32),
            pltpu.VMEM((tok_per_w, L), jnp.float32),
            pltpu.VMEM((tok_per_w, L), jnp.float32),
            pltpu.VMEM((CCH, D), jnp.float32),
            pltpu.VMEM((CCH, D), jnp.float32),
            pltpu.VMEM((CCH, D), jnp.float32),
            pltpu.VMEM((CCH, D), jnp.float32),
            pltpu.SemaphoreType.DMA,
            pltpu.SemaphoreType.DMA,
        ],
    )
    def k(ys_hbm, p0_hbm, p1_hbm, w0_hbm, w1_hbm, out_hbm,
          i0a_v, i1a_v, i0b_v, i1b_v, w0_v, w1_v, r0a, r1a, r0b, r1b, sem, wsem):
        wid = lax.axis_index("s") * NC + lax.axis_index("c")
        base = pl.multiple_of(wid * tok_per_w, 8)
        pltpu.sync_copy(p0_hbm.at[pl.ds(base, CCH)], i0a_v)
        pltpu.sync_copy(p1_hbm.at[pl.ds(base, CCH)], i1a_v)
        pltpu.sync_copy(p0_hbm.at[pl.ds(base + CCH, CCH)], i0b_v)
        pltpu.sync_copy(p1_hbm.at[pl.ds(base + CCH, CCH)], i1b_v)
        g0a = pltpu.async_copy(ys_hbm.at[i0a_v], r0a, sem)
        g1a = pltpu.async_copy(ys_hbm.at[i1a_v], r1a, sem)
        g0b = pltpu.async_copy(ys_hbm.at[i0b_v], r0b, sem)
        g1b = pltpu.async_copy(ys_hbm.at[i1b_v], r1b, sem)
        pltpu.sync_copy(w0_hbm.at[pl.ds(base, tok_per_w)], w0_v)
        pltpu.sync_copy(w1_hbm.at[pl.ds(base, tok_per_w)], w1_v)

        UNROLL = 8
        NBLK = D // (L * UNROLL)

        def weighted_add(r0, r1, coff):
            def row(i, rcarry):
                wa = w0_v[coff + i]
                wb = w1_v[coff + i]

                def blk(j, bcarry):
                    for u in range(UNROLL):
                        sl = pl.ds(j * L * UNROLL + u * L, L)
                        r0[i, sl] = r0[i, sl] * wa + r1[i, sl] * wb
                    return bcarry

                return lax.fori_loop(0, NBLK, blk, rcarry)

            lax.fori_loop(0, CCH, row, 0)

        g0a.wait()
        g1a.wait()
        weighted_add(r0a, r1a, 0)
        wba = pltpu.async_copy(r0a, out_hbm.at[pl.ds(base, CCH)], wsem)
        g0b.wait()
        g1b.wait()
        weighted_add(r0b, r1b, CCH)
        wbb = pltpu.async_copy(r0b, out_hbm.at[pl.ds(base + CCH, CCH)], wsem)
        wba.wait()
        wbb.wait()

    return k


def kernel(x, Wr, W1, b1, W2, b2):
    Bb, Ll, Dd = x.shape
    x_flat = x.reshape(Bb * Ll, Dd)

    idx, w0b, w1b = _router(x_flat, Wr)

    iota_e = jnp.arange(E, dtype=jnp.int32)[None, :]
    oh1 = (idx[:, 0:1] == iota_e).astype(jnp.int32)
    oh2 = (idx[:, 1:2] == iota_e).astype(jnp.int32)
    ohf = oh1 + oh2
    c_incl = jnp.cumsum(ohf, axis=0)
    c_excl = c_incl - ohf
    cnt = c_incl[-1]
    cnt_pad = ((cnt + T - 1) // T) * T
    pad_cum = jnp.cumsum(cnt_pad)
    pad_off = (pad_cum - cnt_pad)[None, :]
    pos0 = jnp.sum(oh1 * (pad_off + c_excl), axis=-1, dtype=jnp.int32)
    pos1 = jnp.sum(oh2 * (pad_off + c_excl + oh1), axis=-1, dtype=jnp.int32)
    block_expert = jnp.minimum(
        jnp.searchsorted(pad_cum, jnp.arange(NB, dtype=jnp.int32) * T, side="right"),
        E - 1,
    ).astype(jnp.int32)
    used_blocks = (pad_cum[-1] // T).astype(jnp.int32)
    block_expert = jnp.concatenate([block_expert, used_blocks[None]])

    Xs = _sc_dispatch_kernel()(x_flat, pos0, pos1)
    Ys = _grouped_mlp(block_expert, Xs, W1, b1, W2, b2)
    out = _sc_combine_kernel()(Ys, pos0, pos1, w0b, w1b)
    return out.reshape(Bb, Ll, Dd)


# speedup vs baseline: 1.1247x; 1.1247x over previous
import functools

import jax
import jax.numpy as jnp
from jax import lax
from jax.experimental import pallas as pl
from jax.experimental.pallas import tpu as pltpu
from jax.experimental.pallas import tpu_sc as plsc

E = 8
K = 2
N = 2048
D = 768
F = 2048
T = 256
P = N * K
S = ((P + E * (T - 1) + T - 1) // T) * T
NB = S // T

NC, NS = 2, 16
NW = NC * NS
L = 16


def _router_body(x_ref, wr_ref, i_ref, w0_ref, w1_ref):
    logits = jnp.dot(x_ref[...], wr_ref[...], preferred_element_type=jnp.float32)
    m = jnp.max(logits, axis=-1, keepdims=True)
    p = jnp.exp(logits - m)
    p = p / jnp.sum(p, axis=-1, keepdims=True)
    iota = lax.broadcasted_iota(jnp.int32, p.shape, 1)
    m1 = jnp.max(p, axis=-1, keepdims=True)
    i1 = jnp.min(jnp.where(p == m1, iota, E), axis=-1, keepdims=True)
    p2 = jnp.where(iota == i1, -1.0, p)
    m2 = jnp.max(p2, axis=-1, keepdims=True)
    i2 = jnp.min(jnp.where(p2 == m2, iota, E), axis=-1, keepdims=True)
    s = m1 + m2
    i_ref[...] = jnp.concatenate([i1, i2], axis=1)
    w0_ref[...] = jnp.broadcast_to(m1 / s, (m1.shape[0], L))
    w1_ref[...] = jnp.broadcast_to(m2 / s, (m2.shape[0], L))


def _router(x_flat, Wr):
    return pl.pallas_call(
        _router_body,
        out_shape=(
            jax.ShapeDtypeStruct((N, K), jnp.int32),
            jax.ShapeDtypeStruct((N, L), jnp.float32),
            jax.ShapeDtypeStruct((N, L), jnp.float32),
        ),
    )(x_flat, Wr)


@functools.cache
def _sc_dispatch_kernel():
    mesh = plsc.VectorSubcoreMesh(
        core_axis_name="c", subcore_axis_name="s", num_cores=NC, num_subcores=NS
    )
    tok_per_w = N // NW

    @functools.partial(
        pl.kernel,
        mesh=mesh,
        out_type=jax.ShapeDtypeStruct((S, D), jnp.float32),
        scratch_types=[
            pltpu.VMEM((tok_per_w, D), jnp.float32),
            pltpu.VMEM((tok_per_w,), jnp.int32),
            pltpu.VMEM((tok_per_w,), jnp.int32),
            pltpu.SemaphoreType.DMA,
            pltpu.SemaphoreType.DMA,
        ],
    )
    def k(x_hbm, p0_hbm, p1_hbm, xs_hbm, xbuf, d0_v, d1_v, lsem, ssem):
        wid = lax.axis_index("s") * NC + lax.axis_index("c")
        tbase = pl.multiple_of(wid * tok_per_w, 8)
        lc = pltpu.async_copy(x_hbm.at[pl.ds(tbase, tok_per_w)], xbuf, lsem)
        pltpu.sync_copy(p0_hbm.at[pl.ds(tbase, tok_per_w)], d0_v)
        pltpu.sync_copy(p1_hbm.at[pl.ds(tbase, tok_per_w)], d1_v)
        lc.wait()
        s0 = pltpu.async_copy(xbuf, xs_hbm.at[d0_v], ssem)
        s1 = pltpu.async_copy(xbuf, xs_hbm.at[d1_v], ssem)
        s0.wait()
        s1.wait()

    return k


def _mlp_body(be_ref, xs_ref, w1_ref, b1_ref, w2_ref, b2_ref, ys_ref):
    i = pl.program_id(0)

    @pl.when(i < be_ref[NB])
    def _():
        h = jax.nn.gelu(
            jnp.dot(xs_ref[...], w1_ref[0], preferred_element_type=jnp.float32)
            + b1_ref[0]
        )
        y = jnp.dot(h, w2_ref[0], preferred_element_type=jnp.float32)
        ys_ref[...] = y + b2_ref[0]


def _grouped_mlp(block_expert, Xs, W1, b1, W2, b2):
    grid_spec = pltpu.PrefetchScalarGridSpec(
        num_scalar_prefetch=1,
        grid=(NB,),
        in_specs=[
            pl.BlockSpec((T, D), lambda i, be: (i, 0)),
            pl.BlockSpec((1, D, F), lambda i, be: (be[i], 0, 0)),
            pl.BlockSpec((1, 1, F), lambda i, be: (be[i], 0, 0)),
            pl.BlockSpec((1, F, D), lambda i, be: (be[i], 0, 0)),
            pl.BlockSpec((1, 1, D), lambda i, be: (be[i], 0, 0)),
        ],
        out_specs=pl.BlockSpec((T, D), lambda i, be: (i, 0)),
    )
    return pl.pallas_call(
        _mlp_body,
        grid_spec=grid_spec,
        out_shape=jax.ShapeDtypeStruct((S, D), jnp.float32),
    )(block_expert, Xs, W1, b1.reshape(E, 1, F), W2, b2.reshape(E, 1, D))


CCH = 32


@functools.cache
def _sc_combine_kernel():
    mesh = plsc.VectorSubcoreMesh(
        core_axis_name="c", subcore_axis_name="s", num_cores=NC, num_subcores=NS
    )
    tok_per_w = N // NW

    @functools.partial(
        pl.kernel,
        mesh=mesh,
        out_type=jax.ShapeDtypeStruct((N, D), jnp.float32),
        scratch_types=[
            pltpu.VMEM((CCH,), jnp.int32),
            pltpu.VMEM((CCH,), jnp.int32),
            pltpu.VMEM((CCH,), jnp.int32),
            pltpu.VMEM((CCH,), jnp.int32),
            pltpu.VMEM((tok_per_w, L), jnp.float32),
            pltpu.VMEM((tok_per_w, L), jnp.float32),
            pltpu.VMEM((CCH, D), jnp.float32),
            pltpu.VMEM((CCH, D), jnp.float32),
            pltpu.VMEM((CCH, D), jnp.float32),
            pltpu.VMEM((CCH, D), jnp.float32),
            pltpu.SemaphoreType.DMA,
            pltpu.SemaphoreType.DMA,
        ],
    )
    def k(ys_hbm, p0_hbm, p1_hbm, w0_hbm, w1_hbm, out_hbm,
          i0a_v, i1a_v, i0b_v, i1b_v, w0_v, w1_v, r0a, r1a, r0b, r1b, sem, wsem):
        wid = lax.axis_index("s") * NC + lax.axis_index("c")
        base = pl.multiple_of(wid * tok_per_w, 8)
        pltpu.sync_copy(p0_hbm.at[pl.ds(base, CCH)], i0a_v)
        pltpu.sync_copy(p1_hbm.at[pl.ds(base, CCH)], i1a_v)
        pltpu.sync_copy(p0_hbm.at[pl.ds(base + CCH, CCH)], i0b_v)
        pltpu.sync_copy(p1_hbm.at[pl.ds(base + CCH, CCH)], i1b_v)
        g0a = pltpu.async_copy(ys_hbm.at[i0a_v], r0a, sem)
        g1a = pltpu.async_copy(ys_hbm.at[i1a_v], r1a, sem)
        g0b = pltpu.async_copy(ys_hbm.at[i0b_v], r0b, sem)
        g1b = pltpu.async_copy(ys_hbm.at[i1b_v], r1b, sem)
        pltpu.sync_copy(w0_hbm.at[pl.ds(base, tok_per_w)], w0_v)
        pltpu.sync_copy(w1_hbm.at[pl.ds(base, tok_per_w)], w1_v)

        def weighted_add(r0, r1, coff):
            def row(i, rcarry):
                wa = w0_v[coff + i]
                wb = w1_v[coff + i]
                for ch in range(D // L):
                    sl = pl.ds(ch * L, L)
                    r0[i, sl] = r0[i, sl] * wa + r1[i, sl] * wb
                return rcarry

            lax.fori_loop(0, CCH, row, 0)

        g0a.wait()
        g1a.wait()
        weighted_add(r0a, r1a, 0)
        wba = pltpu.async_copy(r0a, out_hbm.at[pl.ds(base, CCH)], wsem)
        g0b.wait()
        g1b.wait()
        weighted_add(r0b, r1b, CCH)
        wbb = pltpu.async_copy(r0b, out_hbm.at[pl.ds(base + CCH, CCH)], wsem)
        wba.wait()
        wbb.wait()

    return k


def kernel(x, Wr, W1, b1, W2, b2):
    Bb, Ll, Dd = x.shape
    x_flat = x.reshape(Bb * Ll, Dd)

    idx, w0b, w1b = _router(x_flat, Wr)

    iota_e = jnp.arange(E, dtype=jnp.int32)[None, :]
    oh1 = (idx[:, 0:1] == iota_e).astype(jnp.int32)
    oh2 = (idx[:, 1:2] == iota_e).astype(jnp.int32)
    ohf = oh1 + oh2
    c_incl = jnp.cumsum(ohf, axis=0)
    c_excl = c_incl - ohf
    cnt = c_incl[-1]
    cnt_pad = ((cnt + T - 1) // T) * T
    pad_cum = jnp.cumsum(cnt_pad)
    pad_off = (pad_cum - cnt_pad)[None, :]
    pos0 = jnp.sum(oh1 * (pad_off + c_excl), axis=-1, dtype=jnp.int32)
    pos1 = jnp.sum(oh2 * (pad_off + c_excl + oh1), axis=-1, dtype=jnp.int32)
    block_expert = jnp.minimum(
        jnp.searchsorted(pad_cum, jnp.arange(NB, dtype=jnp.int32) * T, side="right"),
        E - 1,
    ).astype(jnp.int32)
    used_blocks = (pad_cum[-1] // T).astype(jnp.int32)
    block_expert = jnp.concatenate([block_expert, used_blocks[None]])

    Xs = _sc_dispatch_kernel()(x_flat, pos0, pos1)
    Ys = _grouped_mlp(block_expert, Xs, W1, b1, W2, b2)
    out = _sc_combine_kernel()(Ys, pos0, pos1, w0b, w1b)
    return out.reshape(Bb, Ll, Dd)


# speedup vs baseline: 1.1531x; 1.0252x over previous
import functools

import jax
import jax.numpy as jnp
from jax import lax
from jax.experimental import pallas as pl
from jax.experimental.pallas import tpu as pltpu
from jax.experimental.pallas import tpu_sc as plsc

E = 8
K = 2
N = 2048
D = 768
F = 2048
T = 256
P = N * K
S = ((P + E * (T - 1) + T - 1) // T) * T
NB = S // T

NC, NS = 2, 16
NW = NC * NS
L = 16


def _router_body(x_ref, wr_ref, i_ref, w0_ref, w1_ref):
    logits = jnp.dot(x_ref[...], wr_ref[...], preferred_element_type=jnp.float32)
    m = jnp.max(logits, axis=-1, keepdims=True)
    p = jnp.exp(logits - m)
    p = p / jnp.sum(p, axis=-1, keepdims=True)
    iota = lax.broadcasted_iota(jnp.int32, p.shape, 1)
    m1 = jnp.max(p, axis=-1, keepdims=True)
    i1 = jnp.min(jnp.where(p == m1, iota, E), axis=-1, keepdims=True)
    p2 = jnp.where(iota == i1, -1.0, p)
    m2 = jnp.max(p2, axis=-1, keepdims=True)
    i2 = jnp.min(jnp.where(p2 == m2, iota, E), axis=-1, keepdims=True)
    s = m1 + m2
    i_ref[...] = jnp.concatenate([i1, i2], axis=1)
    w0_ref[...] = jnp.broadcast_to(m1 / s, (m1.shape[0], L))
    w1_ref[...] = jnp.broadcast_to(m2 / s, (m2.shape[0], L))


def _router(x_flat, Wr):
    return pl.pallas_call(
        _router_body,
        out_shape=(
            jax.ShapeDtypeStruct((N, K), jnp.int32),
            jax.ShapeDtypeStruct((N, L), jnp.float32),
            jax.ShapeDtypeStruct((N, L), jnp.float32),
        ),
    )(x_flat, Wr)


@functools.cache
def _sc_dispatch_kernel():
    mesh = plsc.VectorSubcoreMesh(
        core_axis_name="c", subcore_axis_name="s", num_cores=NC, num_subcores=NS
    )
    tok_per_w = N // NW

    @functools.partial(
        pl.kernel,
        mesh=mesh,
        out_type=jax.ShapeDtypeStruct((S, D), jnp.float32),
        scratch_types=[
            pltpu.VMEM((tok_per_w, D), jnp.float32),
            pltpu.VMEM((tok_per_w,), jnp.int32),
            pltpu.VMEM((tok_per_w,), jnp.int32),
            pltpu.SemaphoreType.DMA,
            pltpu.SemaphoreType.DMA,
        ],
    )
    def k(x_hbm, p0_hbm, p1_hbm, xs_hbm, xbuf, d0_v, d1_v, lsem, ssem):
        wid = lax.axis_index("s") * NC + lax.axis_index("c")
        tbase = pl.multiple_of(wid * tok_per_w, 8)
        lc = pltpu.async_copy(x_hbm.at[pl.ds(tbase, tok_per_w)], xbuf, lsem)
        c0 = pltpu.async_copy(p0_hbm.at[pl.ds(tbase, tok_per_w)], d0_v, ssem)
        c1 = pltpu.async_copy(p1_hbm.at[pl.ds(tbase, tok_per_w)], d1_v, ssem)
        c0.wait()
        c1.wait()
        lc.wait()
        s0 = pltpu.async_copy(xbuf, xs_hbm.at[d0_v], ssem)
        s1 = pltpu.async_copy(xbuf, xs_hbm.at[d1_v], ssem)
        s0.wait()
        s1.wait()

    return k


def _mlp_body(be_ref, xs_ref, w1_ref, b1_ref, w2_ref, b2_ref, ys_ref):
    i = pl.program_id(0)

    @pl.when(i < be_ref[NB])
    def _():
        h = jax.nn.gelu(
            jnp.dot(xs_ref[...], w1_ref[0], preferred_element_type=jnp.float32)
            + b1_ref[0]
        )
        y = jnp.dot(h, w2_ref[0], preferred_element_type=jnp.float32)
        ys_ref[...] = y + b2_ref[0]


def _grouped_mlp(block_expert, Xs, W1, b1, W2, b2):
    grid_spec = pltpu.PrefetchScalarGridSpec(
        num_scalar_prefetch=1,
        grid=(NB,),
        in_specs=[
            pl.BlockSpec((T, D), lambda i, be: (i, 0)),
            pl.BlockSpec((1, D, F), lambda i, be: (be[i], 0, 0)),
            pl.BlockSpec((1, 1, F), lambda i, be: (be[i], 0, 0)),
            pl.BlockSpec((1, F, D), lambda i, be: (be[i], 0, 0)),
            pl.BlockSpec((1, 1, D), lambda i, be: (be[i], 0, 0)),
        ],
        out_specs=pl.BlockSpec((T, D), lambda i, be: (i, 0)),
    )
    return pl.pallas_call(
        _mlp_body,
        grid_spec=grid_spec,
        out_shape=jax.ShapeDtypeStruct((S, D), jnp.float32),
    )(block_expert, Xs, W1, b1.reshape(E, 1, F), W2, b2.reshape(E, 1, D))


CCH = 32


@functools.cache
def _sc_combine_kernel():
    mesh = plsc.VectorSubcoreMesh(
        core_axis_name="c", subcore_axis_name="s", num_cores=NC, num_subcores=NS
    )
    tok_per_w = N // NW

    @functools.partial(
        pl.kernel,
        mesh=mesh,
        out_type=jax.ShapeDtypeStruct((N, D), jnp.float32),
        scratch_types=[
            pltpu.VMEM((CCH,), jnp.int32),
            pltpu.VMEM((CCH,), jnp.int32),
            pltpu.VMEM((CCH,), jnp.int32),
            pltpu.VMEM((CCH,), jnp.int32),
            pltpu.VMEM((tok_per_w, L), jnp.float32),
            pltpu.VMEM((tok_per_w, L), jnp.float32),
            pltpu.VMEM((CCH, D), jnp.float32),
            pltpu.VMEM((CCH, D), jnp.float32),
            pltpu.VMEM((CCH, D), jnp.float32),
            pltpu.VMEM((CCH, D), jnp.float32),
            pltpu.SemaphoreType.DMA,
            pltpu.SemaphoreType.DMA,
        ],
    )
    def k(ys_hbm, p0_hbm, p1_hbm, w0_hbm, w1_hbm, out_hbm,
          i0a_v, i1a_v, i0b_v, i1b_v, w0_v, w1_v, r0a, r1a, r0b, r1b, sem, wsem):
        wid = lax.axis_index("s") * NC + lax.axis_index("c")
        base = pl.multiple_of(wid * tok_per_w, 8)
        ia0 = pltpu.async_copy(p0_hbm.at[pl.ds(base, CCH)], i0a_v, wsem)
        ia1 = pltpu.async_copy(p1_hbm.at[pl.ds(base, CCH)], i1a_v, wsem)
        ib0 = pltpu.async_copy(p0_hbm.at[pl.ds(base + CCH, CCH)], i0b_v, wsem)
        ib1 = pltpu.async_copy(p1_hbm.at[pl.ds(base + CCH, CCH)], i1b_v, wsem)
        wc0 = pltpu.async_copy(w0_hbm.at[pl.ds(base, tok_per_w)], w0_v, wsem)
        wc1 = pltpu.async_copy(w1_hbm.at[pl.ds(base, tok_per_w)], w1_v, wsem)
        ia0.wait()
        ia1.wait()
        g0a = pltpu.async_copy(ys_hbm.at[i0a_v], r0a, sem)
        g1a = pltpu.async_copy(ys_hbm.at[i1a_v], r1a, sem)
        ib0.wait()
        ib1.wait()
        g0b = pltpu.async_copy(ys_hbm.at[i0b_v], r0b, sem)
        g1b = pltpu.async_copy(ys_hbm.at[i1b_v], r1b, sem)
        wc0.wait()
        wc1.wait()

        def weighted_add(r0, r1, coff):
            def row(i, rcarry):
                wa = w0_v[coff + i]
                wb = w1_v[coff + i]
                for ch in range(D // L):
                    sl = pl.ds(ch * L, L)
                    r0[i, sl] = r0[i, sl] * wa + r1[i, sl] * wb
                return rcarry

            lax.fori_loop(0, CCH, row, 0)

        g0a.wait()
        g1a.wait()
        weighted_add(r0a, r1a, 0)
        wba = pltpu.async_copy(r0a, out_hbm.at[pl.ds(base, CCH)], wsem)
        g0b.wait()
        g1b.wait()
        weighted_add(r0b, r1b, CCH)
        wbb = pltpu.async_copy(r0b, out_hbm.at[pl.ds(base + CCH, CCH)], wsem)
        wba.wait()
        wbb.wait()

    return k


def kernel(x, Wr, W1, b1, W2, b2):
    Bb, Ll, Dd = x.shape
    x_flat = x.reshape(Bb * Ll, Dd)

    idx, w0b, w1b = _router(x_flat, Wr)

    iota_e = jnp.arange(E, dtype=jnp.int32)[None, :]
    oh1 = (idx[:, 0:1] == iota_e).astype(jnp.int32)
    oh2 = (idx[:, 1:2] == iota_e).astype(jnp.int32)
    ohf = oh1 + oh2
    c_incl = jnp.cumsum(ohf, axis=0)
    c_excl = c_incl - ohf
    cnt = c_incl[-1]
    cnt_pad = ((cnt + T - 1) // T) * T
    pad_cum = jnp.cumsum(cnt_pad)
    pad_off = (pad_cum - cnt_pad)[None, :]
    pos0 = jnp.sum(oh1 * (pad_off + c_excl), axis=-1, dtype=jnp.int32)
    pos1 = jnp.sum(oh2 * (pad_off + c_excl + oh1), axis=-1, dtype=jnp.int32)
    block_expert = jnp.minimum(
        jnp.searchsorted(pad_cum, jnp.arange(NB, dtype=jnp.int32) * T, side="right"),
        E - 1,
    ).astype(jnp.int32)
    used_blocks = (pad_cum[-1] // T).astype(jnp.int32)
    block_expert = jnp.concatenate([block_expert, used_blocks[None]])

    Xs = _sc_dispatch_kernel()(x_flat, pos0, pos1)
    Ys = _grouped_mlp(block_expert, Xs, W1, b1, W2, b2)
    out = _sc_combine_kernel()(Ys, pos0, pos1, w0b, w1b)
    return out.reshape(Bb, Ll, Dd)


# speedup vs baseline: 1.2221x; 1.0599x over previous
import functools

import jax
import jax.numpy as jnp
from jax import lax
from jax.experimental import pallas as pl
from jax.experimental.pallas import tpu as pltpu
from jax.experimental.pallas import tpu_sc as plsc

E = 8
K = 2
N = 2048
D = 768
F = 2048
T = 256
P = N * K
S = ((P + E * (T - 1) + T - 1) // T) * T
NB = S // T

NC, NS = 2, 16
NW = NC * NS
L = 16


def _router_body(x_ref, wr_ref, i_ref, w0_ref, w1_ref):
    logits = jnp.dot(x_ref[...], wr_ref[...], preferred_element_type=jnp.float32)
    m = jnp.max(logits, axis=-1, keepdims=True)
    p = jnp.exp(logits - m)
    p = p / jnp.sum(p, axis=-1, keepdims=True)
    iota = lax.broadcasted_iota(jnp.int32, p.shape, 1)
    m1 = jnp.max(p, axis=-1, keepdims=True)
    i1 = jnp.min(jnp.where(p == m1, iota, E), axis=-1, keepdims=True)
    p2 = jnp.where(iota == i1, -1.0, p)
    m2 = jnp.max(p2, axis=-1, keepdims=True)
    i2 = jnp.min(jnp.where(p2 == m2, iota, E), axis=-1, keepdims=True)
    s = m1 + m2
    i_ref[...] = jnp.concatenate([i1, i2], axis=1)
    w0_ref[...] = jnp.broadcast_to(m1 / s, (m1.shape[0], L))
    w1_ref[...] = jnp.broadcast_to(m2 / s, (m2.shape[0], L))


def _router(x_flat, Wr):
    return pl.pallas_call(
        _router_body,
        out_shape=(
            jax.ShapeDtypeStruct((N, K), jnp.int32),
            jax.ShapeDtypeStruct((N, L), jnp.float32),
            jax.ShapeDtypeStruct((N, L), jnp.float32),
        ),
    )(x_flat, Wr)


@functools.cache
def _sc_dispatch_kernel():
    mesh = plsc.VectorSubcoreMesh(
        core_axis_name="c", subcore_axis_name="s", num_cores=NC, num_subcores=NS
    )
    tok_per_w = N // NW

    @functools.partial(
        pl.kernel,
        mesh=mesh,
        out_type=jax.ShapeDtypeStruct((S, D), jnp.float32),
        scratch_types=[
            pltpu.VMEM((tok_per_w, D), jnp.float32),
            pltpu.VMEM((tok_per_w,), jnp.int32),
            pltpu.VMEM((tok_per_w,), jnp.int32),
            pltpu.SemaphoreType.DMA,
            pltpu.SemaphoreType.DMA,
        ],
    )
    def k(x_hbm, p0_hbm, p1_hbm, xs_hbm, xbuf, d0_v, d1_v, lsem, ssem):
        wid = lax.axis_index("s") * NC + lax.axis_index("c")
        tbase = pl.multiple_of(wid * tok_per_w, 8)
        lc = pltpu.async_copy(x_hbm.at[pl.ds(tbase, tok_per_w)], xbuf, lsem)
        c0 = pltpu.async_copy(p0_hbm.at[pl.ds(tbase, tok_per_w)], d0_v, ssem)
        c1 = pltpu.async_copy(p1_hbm.at[pl.ds(tbase, tok_per_w)], d1_v, ssem)
        c0.wait()
        c1.wait()
        lc.wait()
        s0 = pltpu.async_copy(xbuf, xs_hbm.at[d0_v], ssem)
        s1 = pltpu.async_copy(xbuf, xs_hbm.at[d1_v], ssem)
        s0.wait()
        s1.wait()

    return k


def _mlp_body(be_ref, xs_ref, w1_ref, b1_ref, w2_ref, b2_ref, ys_ref, h_ref):
    i = pl.program_id(0)
    used = be_ref[NB]

    @pl.when(jnp.logical_and(i < NB, i < used))
    def _():
        h_ref[i % 2] = jax.nn.gelu(
            jnp.dot(xs_ref[...], w1_ref[0], preferred_element_type=jnp.float32)
            + b1_ref[0]
        )

    @pl.when(jnp.logical_and(i > 0, i <= used))
    def _():
        y = jnp.dot(h_ref[(i + 1) % 2], w2_ref[0],
                    preferred_element_type=jnp.float32)
        ys_ref[...] = y + b2_ref[0]


def _grouped_mlp(block_expert, Xs, W1, b1, W2, b2):
    grid_spec = pltpu.PrefetchScalarGridSpec(
        num_scalar_prefetch=1,
        grid=(NB + 1,),
        in_specs=[
            pl.BlockSpec((T, D), lambda i, be: (jnp.minimum(i, NB - 1), 0)),
            pl.BlockSpec((1, D, F), lambda i, be: (be[jnp.minimum(i, NB - 1)], 0, 0)),
            pl.BlockSpec((1, 1, F), lambda i, be: (be[jnp.minimum(i, NB - 1)], 0, 0)),
            pl.BlockSpec((1, F, D), lambda i, be: (be[jnp.maximum(i - 1, 0)], 0, 0)),
            pl.BlockSpec((1, 1, D), lambda i, be: (be[jnp.maximum(i - 1, 0)], 0, 0)),
        ],
        out_specs=pl.BlockSpec((T, D), lambda i, be: (jnp.maximum(i - 1, 0), 0)),
        scratch_shapes=[pltpu.VMEM((2, T, F), jnp.float32)],
    )
    return pl.pallas_call(
        _mlp_body,
        grid_spec=grid_spec,
        out_shape=jax.ShapeDtypeStruct((S, D), jnp.float32),
    )(block_expert, Xs, W1, b1.reshape(E, 1, F), W2, b2.reshape(E, 1, D))


CCH = 32


@functools.cache
def _sc_combine_kernel():
    mesh = plsc.VectorSubcoreMesh(
        core_axis_name="c", subcore_axis_name="s", num_cores=NC, num_subcores=NS
    )
    tok_per_w = N // NW

    @functools.partial(
        pl.kernel,
        mesh=mesh,
        out_type=jax.ShapeDtypeStruct((N, D), jnp.float32),
        scratch_types=[
            pltpu.VMEM((CCH,), jnp.int32),
            pltpu.VMEM((CCH,), jnp.int32),
            pltpu.VMEM((CCH,), jnp.int32),
            pltpu.VMEM((CCH,), jnp.int32),
            pltpu.VMEM((tok_per_w, L), jnp.float32),
            pltpu.VMEM((tok_per_w, L), jnp.float32),
            pltpu.VMEM((CCH, D), jnp.float32),
            pltpu.VMEM((CCH, D), jnp.float32),
            pltpu.VMEM((CCH, D), jnp.float32),
            pltpu.VMEM((CCH, D), jnp.float32),
            pltpu.SemaphoreType.DMA,
            pltpu.SemaphoreType.DMA,
        ],
    )
    def k(ys_hbm, p0_hbm, p1_hbm, w0_hbm, w1_hbm, out_hbm,
          i0a_v, i1a_v, i0b_v, i1b_v, w0_v, w1_v, r0a, r1a, r0b, r1b, sem, wsem):
        wid = lax.axis_index("s") * NC + lax.axis_index("c")
        base = pl.multiple_of(wid * tok_per_w, 8)
        ia0 = pltpu.async_copy(p0_hbm.at[pl.ds(base, CCH)], i0a_v, wsem)
        ia1 = pltpu.async_copy(p1_hbm.at[pl.ds(base, CCH)], i1a_v, wsem)
        ib0 = pltpu.async_copy(p0_hbm.at[pl.ds(base + CCH, CCH)], i0b_v, wsem)
        ib1 = pltpu.async_copy(p1_hbm.at[pl.ds(base + CCH, CCH)], i1b_v, wsem)
        wc0 = pltpu.async_copy(w0_hbm.at[pl.ds(base, tok_per_w)], w0_v, wsem)
        wc1 = pltpu.async_copy(w1_hbm.at[pl.ds(base, tok_per_w)], w1_v, wsem)
        ia0.wait()
        ia1.wait()
        g0a = pltpu.async_copy(ys_hbm.at[i0a_v], r0a, sem)
        g1a = pltpu.async_copy(ys_hbm.at[i1a_v], r1a, sem)
        ib0.wait()
        ib1.wait()
        g0b = pltpu.async_copy(ys_hbm.at[i0b_v], r0b, sem)
        g1b = pltpu.async_copy(ys_hbm.at[i1b_v], r1b, sem)
        wc0.wait()
        wc1.wait()

        def weighted_add(r0, r1, coff):
            def row(i, rcarry):
                wa = w0_v[coff + i]
                wb = w1_v[coff + i]
                for ch in range(D // L):
                    sl = pl.ds(ch * L, L)
                    r0[i, sl] = r0[i, sl] * wa + r1[i, sl] * wb
                return rcarry

            lax.fori_loop(0, CCH, row, 0)

        g0a.wait()
        g1a.wait()
        weighted_add(r0a, r1a, 0)
        wba = pltpu.async_copy(r0a, out_hbm.at[pl.ds(base, CCH)], wsem)
        g0b.wait()
        g1b.wait()
        weighted_add(r0b, r1b, CCH)
        wbb = pltpu.async_copy(r0b, out_hbm.at[pl.ds(base + CCH, CCH)], wsem)
        wba.wait()
        wbb.wait()

    return k


def kernel(x, Wr, W1, b1, W2, b2):
    Bb, Ll, Dd = x.shape
    x_flat = x.reshape(Bb * Ll, Dd)

    idx, w0b, w1b = _router(x_flat, Wr)

    iota_e = jnp.arange(E, dtype=jnp.int32)[None, :]
    oh1 = (idx[:, 0:1] == iota_e).astype(jnp.int32)
    oh2 = (idx[:, 1:2] == iota_e).astype(jnp.int32)
    ohf = oh1 + oh2
    c_incl = jnp.cumsum(ohf, axis=0)
    c_excl = c_incl - ohf
    cnt = c_incl[-1]
    cnt_pad = ((cnt + T - 1) // T) * T
    pad_cum = jnp.cumsum(cnt_pad)
    pad_off = (pad_cum - cnt_pad)[None, :]
    pos0 = jnp.sum(oh1 * (pad_off + c_excl), axis=-1, dtype=jnp.int32)
    pos1 = jnp.sum(oh2 * (pad_off + c_excl + oh1), axis=-1, dtype=jnp.int32)
    block_expert = jnp.minimum(
        jnp.searchsorted(pad_cum, jnp.arange(NB, dtype=jnp.int32) * T, side="right"),
        E - 1,
    ).astype(jnp.int32)
    used_blocks = (pad_cum[-1] // T).astype(jnp.int32)
    block_expert = jnp.concatenate([block_expert, used_blocks[None]])

    Xs = _sc_dispatch_kernel()(x_flat, pos0, pos1)
    Ys = _grouped_mlp(block_expert, Xs, W1, b1, W2, b2)
    out = _sc_combine_kernel()(Ys, pos0, pos1, w0b, w1b)
    return out.reshape(Bb, Ll, Dd)
